# Initial kernel scaffold; baseline (speedup 1.0000x reference)
import jax
import jax.numpy as jnp
from jax.experimental import pallas as pl


def kernel(x, emb, Wt, bt, Wo, bo):
    raise NotImplementedError("write your pallas kernel here")


# speedup vs baseline: 1.8691x; 1.8691x over previous
import functools

import jax
import jax.numpy as jnp
from jax import lax
from jax.experimental import pallas as pl
from jax.experimental.pallas import tpu as pltpu
from jax.experimental.pallas import tpu_sc as plsc

B = 4096
C = 128
P = 81
NC = 2
NS = 16
NW = NC * NS
RW = B // NW
CB = C // 16


def _h_body(hand_ref, w_ref, bt_ref, bo_ref, o_ref):
    o_ref[...] = (
        jnp.dot(hand_ref[...], w_ref[...], preferred_element_type=jnp.float32)
        + bt_ref[...] + bo_ref[...]
    )


def _hands_encode(hand, w, bt, bo):
    return pl.pallas_call(
        _h_body,
        out_shape=jax.ShapeDtypeStruct((B, C), jnp.float32),
    )(hand, w, bt, bo)


def _sc_body(x_hbm, emb_hbm, h_hbm, out_hbm, emb_v, x_v, h_v, obuf, sem0, sem1):
    cid = lax.axis_index("c")
    sid = lax.axis_index("s")
    wid = sid * NC + cid
    base = wid * RW

    PC = P * C
    pltpu.sync_copy(emb_hbm, emb_v)
    pltpu.sync_copy(x_hbm.at[pl.ds(base * 95, RW * 95)], x_v)
    pltpu.sync_copy(h_hbm.at[pl.ds(base * C, RW * C)], h_v)

    lanes = lax.iota(jnp.int32, 16)
    sems = (sem0, sem1)

    def compute_row(i, slot):
        hvecs = [
            plsc.load_gather(h_v, [i * C + cb * 16 + lanes]) for cb in range(CB)
        ]
        row_off = i * 95
        slot_off = slot * PC

        @pl.loop(0, P)
        def _pos(p):
            idx = plsc.load_gather(x_v, [jnp.full((16,), row_off + p, jnp.int32)])
            src = idx * C + lanes
            dst = slot_off + p * C + lanes
            for cb in range(CB):
                vals = plsc.load_gather(emb_v, [src + cb * 16])
                plsc.store_scatter(obuf, [dst + cb * 16], vals + hvecs[cb])

    @pl.loop(0, RW // 2)
    def _rows(g):
        for slot in range(2):
            i = g * 2 + slot

            @pl.when(g > 0)
            def _drain():
                pltpu.make_async_copy(
                    obuf.at[pl.ds(slot * PC, PC)],
                    out_hbm.at[pl.ds((base + i - 2) * PC, PC)],
                    sems[slot],
                ).wait()

            compute_row(i, slot)
            pltpu.async_copy(
                obuf.at[pl.ds(slot * PC, PC)],
                out_hbm.at[pl.ds((base + i) * PC, PC)],
                sems[slot],
            )

    for slot in range(2):
        pltpu.make_async_copy(
            obuf.at[pl.ds(slot * PC, PC)],
            out_hbm.at[pl.ds((base + RW - 2 + slot) * PC, PC)],
            sems[slot],
        ).wait()


@functools.partial(
    pl.kernel,
    out_type=jax.ShapeDtypeStruct((B * P * C,), jnp.float32),
    mesh=plsc.VectorSubcoreMesh(
        core_axis_name="c", subcore_axis_name="s", num_cores=NC, num_subcores=NS
    ),
    compiler_params=pltpu.CompilerParams(needs_layout_passes=False),
    scratch_types=[
        pltpu.VMEM((88 * C,), jnp.float32),
        pltpu.VMEM((RW * 95,), jnp.int32),
        pltpu.VMEM((RW * C,), jnp.float32),
        pltpu.VMEM((2 * P * C,), jnp.float32),
        pltpu.SemaphoreType.DMA,
        pltpu.SemaphoreType.DMA,
    ],
)
def _sc_gather_add(x_hbm, emb_hbm, h_hbm, out_hbm, emb_v, x_v, h_v, obuf, s0, s1):
    _sc_body(x_hbm, emb_hbm, h_hbm, out_hbm, emb_v, x_v, h_v, obuf, s0, s1)


def kernel(x, emb, Wt, bt, Wo, bo):
    x32 = x.astype(jnp.int32)
    hand = x32[:, 81:95].astype(jnp.float32)
    w = jnp.concatenate([Wt, Wo], axis=1).T
    h = _hands_encode(hand, w, bt, bo)
    out = _sc_gather_add(x32.reshape(-1), emb.reshape(-1), h.reshape(-1))
    return out.reshape(B, P, C)


# speedup vs baseline: 4.1353x; 2.2124x over previous
import functools

import jax
import jax.numpy as jnp
from jax import lax
from jax.experimental import pallas as pl
from jax.experimental.pallas import tpu as pltpu
from jax.experimental.pallas import tpu_sc as plsc

B = 4096
C = 128
P = 81
NC = 2
NS = 16
NW = NC * NS
RW = B // NW
CB = C // 16


def _h_body(hand_ref, w_ref, bt_ref, bo_ref, o_ref):
    o_ref[...] = (
        jnp.dot(hand_ref[...], w_ref[...], preferred_element_type=jnp.float32)
        + bt_ref[...] + bo_ref[...]
    )


def _hands_encode(hand, w, bt, bo):
    return pl.pallas_call(
        _h_body,
        out_shape=jax.ShapeDtypeStruct((B, C), jnp.float32),
    )(hand, w, bt, bo)


def _sc_body(x_hbm, emb_hbm, h_hbm, out_hbm, emb_v, x_v, h_v, obuf, sem0, sem1):
    cid = lax.axis_index("c")
    sid = lax.axis_index("s")
    wid = sid * NC + cid
    base = wid * RW

    PC = P * C
    pltpu.sync_copy(emb_hbm, emb_v)
    pltpu.sync_copy(x_hbm.at[pl.ds(base * 95, RW * 95)], x_v)
    pltpu.sync_copy(h_hbm.at[pl.ds(base * C, RW * C)], h_v)

    lanes = lax.iota(jnp.int32, 16)
    sems = (sem0, sem1)

    def compute_row(i, slot):
        hvecs = [
            plsc.load_gather(h_v, [i * C + cb * 16 + lanes]) for cb in range(CB)
        ]
        row_off = i * 95
        slot_off = slot * PC

        @plsc.parallel_loop(0, P, unroll=3)
        def _pos(p):
            idx = plsc.load_gather(x_v, [jnp.full((16,), row_off + p, jnp.int32)])
            src = idx * C + lanes
            dst = slot_off + p * C + lanes
            for cb in range(CB):
                vals = plsc.load_gather(emb_v, [src + cb * 16])
                plsc.store_scatter(obuf, [dst + cb * 16], vals + hvecs[cb])

    @pl.loop(0, RW // 2)
    def _rows(g):
        for slot in range(2):
            i = g * 2 + slot

            @pl.when(g > 0)
            def _drain():
                pltpu.make_async_copy(
                    obuf.at[pl.ds(slot * PC, PC)],
                    out_hbm.at[pl.ds((base + i - 2) * PC, PC)],
                    sems[slot],
                ).wait()

            compute_row(i, slot)
            pltpu.async_copy(
                obuf.at[pl.ds(slot * PC, PC)],
                out_hbm.at[pl.ds((base + i) * PC, PC)],
                sems[slot],
            )

    for slot in range(2):
        pltpu.make_async_copy(
            obuf.at[pl.ds(slot * PC, PC)],
            out_hbm.at[pl.ds((base + RW - 2 + slot) * PC, PC)],
            sems[slot],
        ).wait()


@functools.partial(
    pl.kernel,
    out_type=jax.ShapeDtypeStruct((B * P * C,), jnp.float32),
    mesh=plsc.VectorSubcoreMesh(
        core_axis_name="c", subcore_axis_name="s", num_cores=NC, num_subcores=NS
    ),
    compiler_params=pltpu.CompilerParams(needs_layout_passes=False),
    scratch_types=[
        pltpu.VMEM((88 * C,), jnp.float32),
        pltpu.VMEM((RW * 95,), jnp.int32),
        pltpu.VMEM((RW * C,), jnp.float32),
        pltpu.VMEM((2 * P * C,), jnp.float32),
        pltpu.SemaphoreType.DMA,
        pltpu.SemaphoreType.DMA,
    ],
)
def _sc_gather_add(x_hbm, emb_hbm, h_hbm, out_hbm, emb_v, x_v, h_v, obuf, s0, s1):
    _sc_body(x_hbm, emb_hbm, h_hbm, out_hbm, emb_v, x_v, h_v, obuf, s0, s1)


def kernel(x, emb, Wt, bt, Wo, bo):
    x32 = x.astype(jnp.int32)
    hand = x32[:, 81:95].astype(jnp.float32)
    w = jnp.concatenate([Wt, Wo], axis=1).T
    h = _hands_encode(hand, w, bt, bo)
    out = _sc_gather_add(x32.reshape(-1), emb.reshape(-1), h.reshape(-1))
    return out.reshape(B, P, C)
